# Initial kernel scaffold; baseline (speedup 1.0000x reference)
import jax
import jax.numpy as jnp
from jax.experimental import pallas as pl


def kernel(neighs, node_count, feature_table):
    raise NotImplementedError("write your pallas kernel here")


# speedup vs baseline: 1.9223x; 1.9223x over previous
import functools

import jax
import jax.numpy as jnp
from jax import lax
from jax.experimental import pallas as pl
from jax.experimental.pallas import tpu as pltpu
from jax.experimental.pallas import tpu_sc as plsc

N_NODES = 10000
NB = 32
D = 128
LANES = 16
VPR = D // LANES

NW = 32
NP = 320
NPAD = NW * NP
CHUNK_NODES = 4
CHUNK_ROWS = CHUNK_NODES * NB
NCHUNK = NP // CHUNK_NODES
IDX_PER_W = NP * NB


def _sc_body(neighs_hbm, dep_hbm, table_hbm, out_hbm,
             idx_v, buf0, buf1, out_v, dep_v, sem0, sem1):
    wid = lax.axis_index("s") * 2 + lax.axis_index("c")
    idx_base = wid * IDX_PER_W
    node_base = wid * NP

    pltpu.sync_copy(neighs_hbm.at[pl.ds(idx_base, IDX_PER_W)], idx_v)
    pltpu.sync_copy(dep_hbm, dep_v)
    dv = dep_v[...]

    bufs = (buf0, buf1)
    sems = (sem0, sem1)

    def gather(g, b):
        pltpu.async_copy(
            table_hbm.at[idx_v.at[pl.ds(g * CHUNK_ROWS, CHUNK_ROWS)]],
            bufs[b], sems[b])

    def wait(g, b):
        pltpu.make_async_copy(
            table_hbm.at[idx_v.at[pl.ds(g * CHUNK_ROWS, CHUNK_ROWS)]],
            bufs[b], sems[b]).wait()

    def reduce_chunk(g, b):
        buf = bufs[b]
        for n in range(CHUNK_NODES):
            def jbody(j, accs, n=n, buf=buf):
                r = n * NB + j * 4
                for u in range(4):
                    accs = tuple(
                        accs[d] + buf[r + u, pl.ds(d * LANES, LANES)]
                        for d in range(VPR))
                return accs
            accs = lax.fori_loop(0, NB // 4, jbody, (dv,) * VPR)
            row = g * CHUNK_NODES + n
            for d in range(VPR):
                out_v[row, pl.ds(d * LANES, LANES)] = accs[d]

    gather(0, 0)
    gather(1, 1)

    def pair(i, _):
        for b in range(2):
            g = 2 * i + b
            wait(g, b)
            reduce_chunk(g, b)

            @pl.when(g + 2 < NCHUNK)
            def _(g=g, b=b):
                gather(g + 2, b)
        return 0

    lax.fori_loop(0, NCHUNK // 2, pair, 0)

    pltpu.sync_copy(out_v, out_hbm.at[pl.ds(node_base, NP)])


@jax.jit
def _sc_sum_aggregate(neighs_pad, dep_arr, feature_table):
    mesh = plsc.VectorSubcoreMesh(core_axis_name="c", subcore_axis_name="s")
    f = functools.partial(
        pl.kernel,
        out_type=jax.ShapeDtypeStruct((NPAD, D), jnp.float32),
        mesh=mesh,
        scratch_types=[
            pltpu.VMEM((IDX_PER_W,), jnp.int32),
            pltpu.VMEM((CHUNK_ROWS, D), jnp.float32),
            pltpu.VMEM((CHUNK_ROWS, D), jnp.float32),
            pltpu.VMEM((NP, D), jnp.float32),
            pltpu.VMEM((LANES,), jnp.float32),
            pltpu.SemaphoreType.DMA,
            pltpu.SemaphoreType.DMA,
        ],
    )(_sc_body)
    return f(neighs_pad, dep_arr, feature_table)


def kernel(neighs, node_count, feature_table):
    dep = (jnp.asarray(node_count) - N_NODES).astype(jnp.float32)
    dep_arr = jnp.full((LANES,), dep, dtype=jnp.float32)
    neighs_pad = jnp.concatenate(
        [neighs, jnp.zeros((NPAD * NB - N_NODES * NB,), dtype=neighs.dtype)])
    out_pad = _sc_sum_aggregate(neighs_pad, dep_arr, feature_table)
    return out_pad[:N_NODES]


# speedup vs baseline: 8.4031x; 4.3715x over previous
import functools

import jax
import jax.numpy as jnp
from jax import lax
from jax.experimental import pallas as pl
from jax.experimental.pallas import tpu as pltpu
from jax.experimental.pallas import tpu_sc as plsc

N_NODES = 10000
NB = 32
D = 128
LANES = 16
VPR = D // LANES

NW = 32
NP = 320
CHUNK_NODES = 4
CHUNK_ROWS = CHUNK_NODES * NB
NCHUNK = NP // CHUNK_NODES
NBUF = 4
IDX_PER_W = NP * NB


def _sc_body(neighs_hbm, dep_hbm, table_hbm, out_hbm,
             idx_v, out_v, dep_v, *bufs_and_sems):
    bufs = bufs_and_sems[:NBUF]
    sems = bufs_and_sems[NBUF:]
    wid = lax.axis_index("s") * 2 + lax.axis_index("c")
    node_base = jnp.minimum(wid * NP, N_NODES - NP)

    pltpu.sync_copy(neighs_hbm.at[pl.ds(node_base * NB, IDX_PER_W)], idx_v)
    pltpu.sync_copy(dep_hbm, dep_v)
    dv = dep_v[...]

    def gather(g, b):
        pltpu.async_copy(
            table_hbm.at[idx_v.at[pl.ds(g * CHUNK_ROWS, CHUNK_ROWS)]],
            bufs[b], sems[b])

    def wait(g, b):
        pltpu.make_async_copy(
            table_hbm.at[idx_v.at[pl.ds(g * CHUNK_ROWS, CHUNK_ROWS)]],
            bufs[b], sems[b]).wait()

    def reduce_chunk(g, b):
        buf = bufs[b]
        for n in range(CHUNK_NODES):
            def jbody(j, accs, buf=buf, n=n):
                r = n * NB + j * 4
                for u in range(4):
                    accs = tuple(
                        accs[d] + buf[r + u, pl.ds(d * LANES, LANES)]
                        for d in range(VPR))
                return accs
            accs = lax.fori_loop(0, NB // 4, jbody, (dv,) * VPR)
            row = g * CHUNK_NODES + n
            for d in range(VPR):
                out_v[row, pl.ds(d * LANES, LANES)] = accs[d]

    for b in range(NBUF):
        gather(b, b)

    def grp(i, _):
        for b in range(NBUF):
            g = i * NBUF + b
            wait(g, b)
            reduce_chunk(g, b)

            @pl.when(g + NBUF < NCHUNK)
            def _(g=g, b=b):
                gather(g + NBUF, b)
        return 0

    lax.fori_loop(0, NCHUNK // NBUF, grp, 0)

    pltpu.sync_copy(out_v, out_hbm.at[pl.ds(node_base, NP)])


@jax.jit
def _sc_sum_aggregate(neighs, dep_arr, feature_table):
    mesh = plsc.VectorSubcoreMesh(core_axis_name="c", subcore_axis_name="s")
    f = functools.partial(
        pl.kernel,
        out_type=jax.ShapeDtypeStruct((N_NODES, D), jnp.float32),
        mesh=mesh,
        scratch_types=[
            pltpu.VMEM((IDX_PER_W,), jnp.int32),
            pltpu.VMEM((NP, D), jnp.float32),
            pltpu.VMEM((LANES,), jnp.float32),
        ] + [pltpu.VMEM((CHUNK_ROWS, D), jnp.float32)] * NBUF
          + [pltpu.SemaphoreType.DMA] * NBUF,
    )(_sc_body)
    return f(neighs, dep_arr, feature_table)


def kernel(neighs, node_count, feature_table):
    dep = (jnp.asarray(node_count) - N_NODES).astype(jnp.float32)
    dep_arr = jnp.full((LANES,), dep, dtype=jnp.float32)
    return _sc_sum_aggregate(neighs, dep_arr, feature_table)
